# baseline (speedup 1.0000x reference)
import jax
import jax.numpy as jnp
from jax import lax
from jax.experimental import pallas as pl
from jax.experimental.pallas import tpu as pltpu


def _body(sb_ref, x_ref, o_ref):
    scale = sb_ref[0, :, 0:1]
    bias = sb_ref[0, :, 1:2]
    o_ref[...] = x_ref[...] * scale + bias


def kernel(inputs, mask, alpha, beta):
    B, C = inputs.shape
    xt = inputs.T
    col = lax.broadcasted_iota(jnp.int32, (C, 2), 1)
    sb = jnp.where(
        mask[:, None],
        jnp.where(col == 0, alpha[0], beta[0]),
        jnp.where(col == 0, 1.0, 0.0),
    ).astype(jnp.float32)
    blk = 2048
    sb3 = jnp.broadcast_to(sb[None], (B // blk, C, 2))
    out_t = pl.pallas_call(
        _body,
        grid=(B // blk,),
        in_specs=[
            pl.BlockSpec((1, C, 2), lambda i: (i, 0, 0)),
            pl.BlockSpec((C, blk), lambda i: (0, i)),
        ],
        out_specs=pl.BlockSpec((C, blk), lambda i: (0, i)),
        out_shape=jax.ShapeDtypeStruct((C, B), jnp.float32),
    )(sb3, xt)
    return out_t.T


# speedup vs baseline: 1.0595x; 1.0595x over previous
import jax
import jax.numpy as jnp
from jax import lax
from jax.experimental import pallas as pl
from jax.experimental.pallas import tpu as pltpu


def _body(sb_ref, x_ref, o_ref):
    scale = sb_ref[:, 0:1]
    bias = sb_ref[:, 1:2]
    o_ref[...] = x_ref[...] * scale + bias


def kernel(inputs, mask, alpha, beta):
    B, C = inputs.shape
    xt = inputs.T
    col = lax.broadcasted_iota(jnp.int32, (C, 2), 1)
    sb = jnp.where(
        mask[:, None],
        jnp.where(col == 0, alpha[0], beta[0]),
        jnp.where(col == 0, 1.0, 0.0),
    ).astype(jnp.float32)
    blk = 2048
    out_t = pl.pallas_call(
        _body,
        grid=(B // blk,),
        in_specs=[
            pl.BlockSpec((C, 2), lambda i: (0, 0)),
            pl.BlockSpec((C, blk), lambda i: (0, i)),
        ],
        out_specs=pl.BlockSpec((C, blk), lambda i: (0, i)),
        out_shape=jax.ShapeDtypeStruct((C, B), jnp.float32),
    )(sb, xt)
    return out_t.T
